# Initial kernel scaffold; baseline (speedup 1.0000x reference)
import jax
import jax.numpy as jnp
from jax.experimental import pallas as pl


def kernel(history_tokens, history_post_tokens, history_author_tokens, history_action_tokens, history_time_gap, history_group_ids, history_mask, token_table, time_table, group_table, pos_table, ln_gamma, ln_beta, W1, b1, W2, b2, sep_token):
    raise NotImplementedError("write your pallas kernel here")


# speedup vs baseline: 2.1141x; 2.1141x over previous
import functools

import jax
import jax.numpy as jnp
from jax import lax
from jax.experimental import pallas as pl
from jax.experimental.pallas import tpu as pltpu
from jax.experimental.pallas import tpu_sc as plsc

_B, _L, _T, _H = 16, 2048, 4096, 256
_NF = _B * _L
_CH = 128
_NW = 32
_BT = 256
_NTB = _T // _BT
_LB = _L // _BT
_D6 = 6 * _H
_DH = 4 * _H


def _sc_gather6(tok_tbl, time_tbl, grp_tbl, ids):
    per_w = _NF // _NW
    nch = per_w // _CH
    mesh = plsc.VectorSubcoreMesh(core_axis_name="c", subcore_axis_name="s")
    out_t = tuple(jax.ShapeDtypeStruct((_NF, _H), jnp.float32)
                  for _ in range(6))

    @functools.partial(
        pl.kernel, mesh=mesh, out_type=out_t,
        scratch_types=[pltpu.VMEM((_CH,), jnp.int32),
                       pltpu.VMEM((_CH, _H), jnp.float32),
                       pltpu.SemaphoreType.DMA])
    def k(tt, mt, gt, i0, i1, i2, i3, i4, i5,
          o0, o1, o2, o3, o4, o5, idx_v, rows_v, sem):
        wid = lax.axis_index("s") * 2 + lax.axis_index("c")
        base = wid * per_w
        for tbl, isrc, dst in ((tt, i0, o0), (tt, i1, o1), (tt, i2, o2),
                               (tt, i3, o3), (mt, i4, o4), (gt, i5, o5)):
            def body(c, carry, tbl=tbl, isrc=isrc, dst=dst):
                off = base + c * _CH
                pltpu.sync_copy(isrc.at[pl.ds(off, _CH)], idx_v)
                pltpu.async_copy(tbl.at[idx_v], rows_v, sem).wait()
                pltpu.sync_copy(rows_v, dst.at[pl.ds(off, _CH)])
                return carry
            lax.fori_loop(0, nch, body, 0)

    return k(tok_tbl, time_tbl, grp_tbl, *ids)


def _mlp_body(n_ref, x0, x1, x2, x3, x4, x5, g_ref, be_ref,
              w1_ref, b1_ref, w2_ref, b2_ref, o_ref):
    b = pl.program_id(0)
    i = pl.program_id(1)
    nb = n_ref[b]

    @pl.when(i * _BT < nb)
    def _compute():
        x = jnp.concatenate([x0[0], x1[0], x2[0], x3[0], x4[0], x5[0]],
                            axis=-1)
        mu = jnp.mean(x, axis=1, keepdims=True)
        var = jnp.mean(x * x, axis=1, keepdims=True) - mu * mu
        xn = (x - mu) * lax.rsqrt(var + 1e-5)
        xn = xn * g_ref[0] + be_ref[0]
        h = jnp.dot(xn.astype(jnp.bfloat16), w1_ref[...],
                    preferred_element_type=jnp.float32) + b1_ref[0]
        a = h * jax.nn.sigmoid(h)
        o = jnp.dot(a.astype(jnp.bfloat16), w2_ref[...],
                    preferred_element_type=jnp.float32) + b2_ref[0]
        o_ref[0] = o.astype(jnp.bfloat16)

    @pl.when(i * _BT >= nb)
    def _zero():
        o_ref[...] = jnp.zeros_like(o_ref)


def _mlp(xs, n_arr, gamma, beta, w1t, b1, w2t, b2):
    grid_spec = pltpu.PrefetchScalarGridSpec(
        num_scalar_prefetch=1,
        grid=(_B, _LB),
        in_specs=[pl.BlockSpec((1, _BT, _H), lambda b, i, n: (b, i, 0))
                  for _ in range(6)] + [
            pl.BlockSpec((1, 1, _D6), lambda b, i, n: (0, 0, 0)),
            pl.BlockSpec((1, 1, _D6), lambda b, i, n: (0, 0, 0)),
            pl.BlockSpec((_D6, _DH), lambda b, i, n: (0, 0)),
            pl.BlockSpec((1, 1, _DH), lambda b, i, n: (0, 0, 0)),
            pl.BlockSpec((_DH, _H), lambda b, i, n: (0, 0)),
            pl.BlockSpec((1, 1, _H), lambda b, i, n: (0, 0, 0)),
        ],
        out_specs=pl.BlockSpec((1, _BT, _H), lambda b, i, n: (b, i, 0)),
    )
    return pl.pallas_call(
        _mlp_body, grid_spec=grid_spec,
        out_shape=jax.ShapeDtypeStruct((_B, _L, _H), jnp.bfloat16),
    )(n_arr, *xs, gamma, beta, w1t, b1, w2t, b2)


def _merge_body(j_ref, j2_ref, evA, evB, p_ref, s_ref, pos_ref, sep_ref,
                o_ref):
    b = pl.program_id(0)
    t = pl.program_id(1)
    j = j_ref[b * _NTB + t]
    W = 4 * _BT
    io0 = lax.broadcasted_iota(jnp.int32, (W, _BT), 0)
    io1 = lax.broadcasted_iota(jnp.int32, (W, _BT), 1)
    pid_b = jnp.broadcast_to(p_ref[0], (W, _BT))
    sl_b = jnp.broadcast_to(s_ref[0], (W, _BT))
    local = pid_b - j * _BT
    oh_ev = (io0 == local) & (sl_b == 1)
    oh_sep = (io0 == 2 * _BT) & (sl_b == 2)
    oh_pos = (io0 - 3 * _BT == io1) & (sl_b != 0)
    ohT = (oh_ev | oh_sep | oh_pos).astype(jnp.bfloat16)
    win = jnp.concatenate([evA[0], evB[0], sep_ref[0], pos_ref[...]],
                          axis=0)
    o_ref[0] = lax.dot_general(
        ohT, win, dimension_numbers=(((0,), (0,)), ((), ())),
        preferred_element_type=jnp.float32)


def _merge(ev, pidx3, sel3, j_arr, j2_arr, pos_tab, sep3):
    grid_spec = pltpu.PrefetchScalarGridSpec(
        num_scalar_prefetch=2,
        grid=(_B, _NTB),
        in_specs=[
            pl.BlockSpec((1, _BT, _H),
                         lambda b, t, j, j2: (b, j[b * _NTB + t], 0)),
            pl.BlockSpec((1, _BT, _H),
                         lambda b, t, j, j2: (b, j2[b * _NTB + t], 0)),
            pl.BlockSpec((1, 1, _BT),
                         lambda b, t, j, j2: (b * _NTB + t, 0, 0)),
            pl.BlockSpec((1, 1, _BT),
                         lambda b, t, j, j2: (b * _NTB + t, 0, 0)),
            pl.BlockSpec((_BT, _H), lambda b, t, j, j2: (t, 0)),
            pl.BlockSpec((1, _BT, _H), lambda b, t, j, j2: (0, 0, 0)),
        ],
        out_specs=pl.BlockSpec((1, _BT, _H), lambda b, t, j, j2: (b, t, 0)),
    )
    return pl.pallas_call(
        _merge_body, grid_spec=grid_spec,
        out_shape=jax.ShapeDtypeStruct((_B, _T, _H), jnp.float32),
    )(j_arr, j2_arr, ev, ev, pidx3, sel3, pos_tab, sep3)


def kernel(history_tokens, history_post_tokens, history_author_tokens,
           history_action_tokens, history_time_gap, history_group_ids,
           history_mask, token_table, time_table, group_table, pos_table,
           ln_gamma, ln_beta, W1, b1, W2, b2, sep_token):
    i32 = jnp.int32
    mask = history_mask.astype(bool)
    group = history_group_ids.astype(i32)

    idx = jnp.arange(_L, dtype=i32)
    a = jnp.where(mask, idx[None, :], _L)
    rev_min = lax.cummin(a[:, ::-1], axis=1)[:, ::-1]
    nv = jnp.concatenate(
        [rev_min[:, 1:], jnp.full((_B, 1), _L, dtype=a.dtype)], axis=1)
    has_next = nv < _L
    g_next = jnp.take_along_axis(group, jnp.clip(nv, 0, _L - 1), axis=1)
    sep_after = mask & has_next & (group != g_next)
    c = mask.astype(i32) + sep_after.astype(i32)
    total = jnp.sum(c, axis=1, keepdims=True)
    off = jnp.cumsum(c, axis=1) - c
    pos_ev = _T - total + off
    pos_ev = jnp.where(mask, pos_ev, _T)
    pos_sep = jnp.where(sep_after, pos_ev + 1, _T)
    bi = jnp.arange(_B, dtype=i32)[:, None]
    gather_l = jnp.zeros((_B, _T), dtype=i32).at[bi, pos_ev].set(
        jnp.broadcast_to(idx[None, :], (_B, _L)), mode='drop')
    sel = jnp.zeros((_B, _T), dtype=i32)
    sel = sel.at[bi, pos_ev].set(1, mode='drop')
    sel = sel.at[bi, pos_sep].set(2, mode='drop')

    mi = mask.astype(i32)
    pc = jnp.cumsum(mi, axis=1) - 1
    n_arr = jnp.sum(mi, axis=1).astype(i32)
    packed_l = jnp.zeros((_B, _L), dtype=i32).at[
        bi, jnp.where(mask, pc, _L)].set(
        jnp.broadcast_to(idx[None, :], (_B, _L)), mode='drop')
    pidx = jnp.take_along_axis(pc, gather_l, axis=1)
    pidx = jnp.where(sel == 1, pidx, -1)

    big = jnp.int32(1 << 30)
    p4 = pidx.reshape(_B, _NTB, _BT)
    w0 = jnp.min(jnp.where(p4 >= 0, p4, big), axis=2)
    j_arr = jnp.clip(jnp.where(w0 >= big, 0, w0 // _BT), 0, _LB - 1)
    j2_arr = jnp.minimum(j_arr + 1, _LB - 1)
    j_arr = j_arr.reshape(-1).astype(i32)
    j2_arr = j2_arr.reshape(-1).astype(i32)

    def packed_ids(arr):
        return jnp.take_along_axis(arr.astype(i32), packed_l,
                                   axis=1).reshape(_NF)

    ids = [packed_ids(history_tokens), packed_ids(history_post_tokens),
           packed_ids(history_author_tokens), packed_ids(history_action_tokens),
           packed_ids(jnp.clip(history_time_gap, 0, 128)), packed_ids(group)]

    xs = _sc_gather6(token_table, time_table, group_table, ids)
    xs = [x.reshape(_B, _L, _H) for x in xs]

    gamma = ln_gamma.reshape(1, 1, _D6)
    beta = ln_beta.reshape(1, 1, _D6)
    w1t = W1.T.astype(jnp.bfloat16)
    w2t = W2.T.astype(jnp.bfloat16)
    ev = _mlp(xs, n_arr, gamma, beta, w1t,
              b1.reshape(1, 1, _DH), w2t, b2.reshape(1, 1, _H))

    pidx3 = pidx.reshape(_B * _NTB, 1, _BT)
    sel3 = sel.reshape(_B * _NTB, 1, _BT)
    sep_pad = jnp.zeros((1, _BT, _H), jnp.bfloat16).at[0, 0].set(
        sep_token.astype(jnp.bfloat16))
    merged = _merge(ev, pidx3, sel3, j_arr, j2_arr,
                    pos_table.astype(jnp.bfloat16), sep_pad)
    return merged, sel != 0
